# baseline (speedup 1.0000x reference)
import jax
import jax.numpy as jnp
from jax.experimental import pallas as pl
from jax.experimental.pallas import tpu as pltpu

_SEQ = 32896
_N_EDGE = 32640
_GB = 8
_NJ = 5
_EW = _N_EDGE // _NJ


def _onehot_kernel(x_ref, node_ref, edge_ref):
    j = pl.program_id(1)

    xe = x_ref[0, :, pl.ds(256 + j * _EW, _EW)].astype(jnp.float32)
    j_row = jax.lax.broadcasted_iota(jnp.int32, (_GB * 8, _GB), 0)
    j_col = jax.lax.broadcasted_iota(jnp.int32, (_GB * 8, _GB), 1)
    rep = (j_col == j_row // 8).astype(jnp.float32)
    r = jax.lax.dot(rep, xe, preferred_element_type=jnp.float32)
    vrow = (
        jax.lax.broadcasted_iota(jnp.int32, (_GB * 8, 1), 0) % 8
    ).astype(jnp.float32)
    edge = jnp.where(r == vrow, 100.0, -100.0)
    edge_ref[...] = edge.reshape(_GB, 8, _EW)

    @pl.when(j == 0)
    def _():
        lane = jax.lax.broadcasted_iota(jnp.int32, (1, 128), 1).astype(
            jnp.float32
        )
        ones = jnp.ones((1, 128), dtype=jnp.float32)
        for k in range(_GB):
            xn = x_ref[0, k : k + 1, :256].astype(jnp.float32)
            col = jax.lax.dot_general(
                xn,
                ones,
                dimension_numbers=(((0,), (0,)), ((), ())),
                preferred_element_type=jnp.float32,
            )
            node_ref[k] = jnp.where(col == lane, 100.0, -100.0)


def kernel(tokens, pad_mask, t, x0):
    B = x0.shape[0]
    xr = x0.reshape(B // _GB, _GB, _SEQ)
    node, edge_vm = pl.pallas_call(
        _onehot_kernel,
        grid=(B // _GB, _NJ),
        in_specs=[pl.BlockSpec((1, _GB, _SEQ), lambda i, j: (i, 0, 0))],
        out_specs=[
            pl.BlockSpec((_GB, 256, 128), lambda i, j: (i, 0, 0)),
            pl.BlockSpec((_GB, 8, _EW), lambda i, j: (i, 0, j)),
        ],
        out_shape=[
            jax.ShapeDtypeStruct((B, 256, 128), jnp.float32),
            jax.ShapeDtypeStruct((B, 8, _N_EDGE), jnp.float32),
        ],
        compiler_params=pltpu.CompilerParams(
            dimension_semantics=("parallel", "arbitrary")
        ),
    )(xr)
    return node, edge_vm.transpose(0, 2, 1)


# speedup vs baseline: 1.6053x; 1.6053x over previous
import jax
import jax.numpy as jnp
from jax.experimental import pallas as pl

_SEQ = 32896
_N_EDGE = 32640
_GB = 8


def _onehot_kernel(x_ref, node_ref, edge_ref):
    xf = x_ref[0].astype(jnp.float32)

    j_row = jax.lax.broadcasted_iota(jnp.int32, (_GB * 8, _GB), 0)
    j_col = jax.lax.broadcasted_iota(jnp.int32, (_GB * 8, _GB), 1)
    rep = (j_col == j_row // 8).astype(jnp.float32)
    r = jax.lax.dot(rep, xf, preferred_element_type=jnp.float32)
    vrow = (
        jax.lax.broadcasted_iota(jnp.int32, (_GB * 8, 1), 0) % 8
    ).astype(jnp.float32)
    edge = jnp.where(r[:, 256:] == vrow, 100.0, -100.0)
    edge_ref[...] = edge.reshape(_GB, 8, _N_EDGE)

    lane = jax.lax.broadcasted_iota(jnp.int32, (1, 128), 1).astype(jnp.float32)
    ones = jnp.ones((1, 128), dtype=jnp.float32)
    for k in range(_GB):
        xn = xf[k : k + 1, :256]
        col = jax.lax.dot_general(
            xn,
            ones,
            dimension_numbers=(((0,), (0,)), ((), ())),
            preferred_element_type=jnp.float32,
        )
        node_ref[k] = jnp.where(col == lane, 100.0, -100.0)


def kernel(tokens, pad_mask, t, x0):
    B = x0.shape[0]
    xr = x0.reshape(B // _GB, _GB, _SEQ)
    node, edge_vm = pl.pallas_call(
        _onehot_kernel,
        grid=(B // _GB,),
        in_specs=[pl.BlockSpec((1, _GB, _SEQ), lambda i: (i, 0, 0))],
        out_specs=[
            pl.BlockSpec((_GB, 256, 128), lambda i: (i, 0, 0)),
            pl.BlockSpec((_GB, 8, _N_EDGE), lambda i: (i, 0, 0)),
        ],
        out_shape=[
            jax.ShapeDtypeStruct((B, 256, 128), jnp.float32),
            jax.ShapeDtypeStruct((B, 8, _N_EDGE), jnp.float32),
        ],
    )(xr)
    return node, edge_vm.transpose(0, 2, 1)
